# baseline (speedup 1.0000x reference)
import functools

import jax
import jax.numpy as jnp
from jax import lax
from jax.experimental import pallas as pl
from jax.experimental.pallas import tpu as pltpu
from jax.experimental.pallas import tpu_sc as plsc

_B = 16384
_D = 200 * 64
_NC, _NS = 2, 16
_NW = _NC * _NS
_ROWS_PER_W = _B // _NW
_R = 8
_STEPS = _ROWS_PER_W // _R
_K = 4


def _sc_body(tab_hbm, out_hbm, tab_v, sem):
    c = lax.axis_index("c")
    s = lax.axis_index("s")
    wid = s * _NC + c
    base = wid * _ROWS_PER_W
    for r in range(_R):
        pltpu.sync_copy(tab_hbm, tab_v.at[r])

    for j in range(_K):
        pltpu.make_async_copy(
            tab_v, out_hbm.at[pl.ds(base + j * _R, _R)], sem).start()

    def step(i, carry):
        pltpu.make_async_copy(
            tab_v, out_hbm.at[pl.ds(base + (i + _K) * _R, _R)], sem).start()
        pltpu.make_async_copy(
            tab_v, out_hbm.at[pl.ds(base, _R)], sem).wait()
        return carry

    lax.fori_loop(0, _STEPS - _K, step, 0)
    for j in range(_K):
        pltpu.make_async_copy(
            tab_v, out_hbm.at[pl.ds(base, _R)], sem).wait()


_sc_bcast = functools.partial(
    pl.kernel,
    out_type=jax.ShapeDtypeStruct((_B, _D), jnp.float32),
    mesh=plsc.VectorSubcoreMesh(core_axis_name="c", subcore_axis_name="s"),
    scratch_types=[
        pltpu.VMEM((_R, _D), jnp.float32),
        pltpu.SemaphoreType.DMA,
    ],
)(_sc_body)


def _bcast_body(tab_ref, out_ref):
    out_ref[...] = jnp.broadcast_to(tab_ref[...], out_ref.shape)


def _tc_bcast(pos_emb, bb):
    tab = pos_emb.reshape(1, _D)
    return pl.pallas_call(
        _bcast_body,
        grid=(_B // bb,),
        in_specs=[pl.BlockSpec((1, _D), lambda i: (0, 0))],
        out_specs=pl.BlockSpec((bb, _D), lambda i: (i, 0)),
        out_shape=jax.ShapeDtypeStruct((_B, _D), jnp.float32),
    )(tab)


def kernel(x, pos_emb):
    out = _sc_bcast(pos_emb.reshape(_D))
    return out.reshape(_B, 200, 64)


# speedup vs baseline: 1.0414x; 1.0414x over previous
import functools

import jax
import jax.numpy as jnp
from jax import lax
from jax.experimental import pallas as pl
from jax.experimental.pallas import tpu as pltpu
from jax.experimental.pallas import tpu_sc as plsc

_B = 16384
_D = 200 * 64
_NC, _NS = 2, 16
_NW = _NC * _NS
_ROWS_PER_W = _B // _NW
_R = 8
_STEPS = _ROWS_PER_W // _R
_K = 4


def _sc_body(tab_hbm, out_hbm, tab_v, sem):
    c = lax.axis_index("c")
    s = lax.axis_index("s")
    wid = s * _NC + c
    base = wid * _ROWS_PER_W
    for r in range(_R):
        pltpu.sync_copy(tab_hbm, tab_v.at[r])

    for j in range(_K):
        pltpu.make_async_copy(
            tab_v, out_hbm.at[pl.ds(base + j * _R, _R)], sem).start()

    def step(i, carry):
        pltpu.make_async_copy(
            tab_v, out_hbm.at[pl.ds(base + (i + _K) * _R, _R)], sem).start()
        pltpu.make_async_copy(
            tab_v, out_hbm.at[pl.ds(base, _R)], sem).wait()
        return carry

    lax.fori_loop(0, _STEPS - _K, step, 0)
    for j in range(_K):
        pltpu.make_async_copy(
            tab_v, out_hbm.at[pl.ds(base, _R)], sem).wait()


_sc_bcast = functools.partial(
    pl.kernel,
    out_type=jax.ShapeDtypeStruct((_B, _D), jnp.float32),
    mesh=plsc.VectorSubcoreMesh(core_axis_name="c", subcore_axis_name="s"),
    scratch_types=[
        pltpu.VMEM((_R, _D), jnp.float32),
        pltpu.SemaphoreType.DMA,
    ],
)(_sc_body)


def _bcast_body(tab_ref, out_ref):
    out_ref[...] = jnp.broadcast_to(tab_ref[...], out_ref.shape)


def _tc_bcast(pos_emb, bb):
    tab = pos_emb.reshape(1, _D)
    return pl.pallas_call(
        _bcast_body,
        grid=(_B // bb,),
        in_specs=[pl.BlockSpec((1, _D), lambda i: (0, 0))],
        out_specs=pl.BlockSpec((bb, _D), lambda i: (i, 0)),
        out_shape=jax.ShapeDtypeStruct((_B, _D), jnp.float32),
    )(tab)


def kernel(x, pos_emb):
    out = _tc_bcast(pos_emb, 128)
    return out.reshape(_B, 200, 64)


# speedup vs baseline: 2.9689x; 2.8508x over previous
import functools

import jax
import jax.numpy as jnp
from jax import lax
from jax.experimental import pallas as pl
from jax.experimental.pallas import tpu as pltpu
from jax.experimental.pallas import tpu_sc as plsc

_B = 16384
_S = 200
_E = 64
_D = _S * _E
_NC, _NS = 2, 16
_NW = _NC * _NS
_SW = 512
_NSL = _B // _SW
_MAXK = -(-_S // _NW)


def _splat_body(tab_ref, out_ref):
    out_ref[...] = jnp.broadcast_to(tab_ref[...], out_ref.shape)


def _tc_splat(tab_col):
    return pl.pallas_call(
        _splat_body,
        grid=(8,),
        in_specs=[pl.BlockSpec((_D // 8, 1), lambda i: (i, 0))],
        out_specs=pl.BlockSpec((_D // 8, _SW), lambda i: (i, 0)),
        out_shape=jax.ShapeDtypeStruct((_D, _SW), jnp.float32),
    )(tab_col)


def _sc_body_t(tab512_hbm, out_hbm, slab0, slab1, sem0, sem1):
    c = lax.axis_index("c")
    sid = lax.axis_index("s")
    w = sid * _NC + c
    slabs = (slab0, slab1)
    sems = (sem0, sem1)

    def build(slab, s_row):
        pltpu.sync_copy(tab512_hbm.at[pl.ds(s_row * _E, _E), :], slab)

    def fire(slab, sem, s_row):
        for j in range(_NSL):
            pltpu.make_async_copy(
                slab, out_hbm.at[s_row, :, pl.ds(j * _SW, _SW)], sem).start()

    def drain(slab, sem):
        for j in range(_NSL):
            pltpu.make_async_copy(
                slab, out_hbm.at[0, :, pl.ds(j * _SW, _SW)], sem).wait()

    for k in range(_MAXK):
        p = k % 2
        s_row = w + _NW * k
        if k >= 2:
            drain(slabs[p], sems[p])
        if k < _MAXK - 1:
            build(slabs[p], s_row)
            fire(slabs[p], sems[p], s_row)
        else:
            @pl.when(s_row < _S)
            def _():
                build(slabs[p], s_row)
                fire(slabs[p], sems[p], s_row)

    @pl.when(w + _NW * (_MAXK - 1) < _S)
    def _():
        drain(slabs[(_MAXK - 1) % 2], sems[(_MAXK - 1) % 2])
    drain(slabs[(_MAXK - 2) % 2], sems[(_MAXK - 2) % 2])


_sc_bcast_t = functools.partial(
    pl.kernel,
    out_type=jax.ShapeDtypeStruct((_S, _E, _B), jnp.float32),
    mesh=plsc.VectorSubcoreMesh(core_axis_name="c", subcore_axis_name="s"),
    scratch_types=[
        pltpu.VMEM((_E, _SW), jnp.float32),
        pltpu.VMEM((_E, _SW), jnp.float32),
        pltpu.SemaphoreType.DMA,
        pltpu.SemaphoreType.DMA,
    ],
)(_sc_body_t)


def kernel(x, pos_emb):
    tab512 = _tc_splat(pos_emb.reshape(_D, 1))
    t = _sc_bcast_t(tab512)
    return jnp.transpose(t, (2, 0, 1))
